# baseline (speedup 1.0000x reference)
import functools

import jax
import jax.numpy as jnp
from jax import lax
from jax.experimental import pallas as pl
from jax.experimental.pallas import tpu as pltpu
from jax.experimental.pallas import tpu_sc as plsc

B = 16384
EMB = 16
L = 16
NC = 2
NS = 16
I_PER = B // (NC * NS)
CH = 128

N_USR = 1000001
N_MOV = 100001
NSEG = 6256
N_MOV_PAD = NS * NSEG
BLK = 131072
GRID = -(-N_USR // BLK)
N_PAD = GRID * BLK


def _sc_scatter(yv, fcw, s_out,
                yidx, w_buf, zero_buf, acc):
    c = lax.axis_index("c")
    s = lax.axis_index("s")
    wid = s * NC + c
    jbase = wid * I_PER

    zero = jnp.zeros((L,), jnp.float32)

    def zbody(b, carry):
        zero_buf[pl.ds(b * L, L)] = zero
        return carry

    lax.fori_loop(0, NSEG // L, zbody, 0)
    pltpu.sync_copy(zero_buf, acc.at[pl.ds(s * NSEG, NSEG)])
    plsc.subcore_barrier()

    for i in range(I_PER // CH):
        pltpu.sync_copy(yv.at[pl.ds(jbase + i * CH, CH)], yidx.at[i])
        pltpu.sync_copy(fcw.at[pl.ds(jbase + i * CH, CH)], w_buf.at[i])
        pltpu.sync_copy(w_buf.at[i], acc.at[yidx.at[i]], add=True)
    plsc.subcore_barrier()
    pltpu.sync_copy(acc.at[pl.ds(s * NSEG, NSEG)],
                    s_out.at[pl.ds(c * N_MOV_PAD + s * NSEG, NSEG)])


def _tc_user(ue_ref, s_ref, me_ref, out_ref, vw_out, vw_scr):
    @pl.when(pl.program_id(0) == 0)
    def _():
        s_sum = s_ref[0:N_MOV_PAD] + s_ref[N_MOV_PAD:2 * N_MOV_PAD]
        w_tot = jnp.sum(s_sum)
        v = jnp.sum(me_ref[...] * s_sum[0:N_MOV][None, :], axis=1)
        vw_scr[0, 0:EMB] = v
        vw_scr[0, EMB:2 * EMB] = jnp.zeros((EMB,), jnp.float32) + w_tot
        vw_scr[0, 2 * EMB:128] = jnp.zeros((128 - 2 * EMB,), jnp.float32)

    u = jax.lax.dot_general(vw_scr[0:1, 0:EMB], ue_ref[...],
                            (((1,), (0,)), ((), ())))
    out_ref[...] = u[0]
    vw_out[...] = vw_scr[...]


def _sc_out(u_hbm, ub_hbm, mb_hbm, vw_hbm, xv, yv, fcb,
            out_hbm,
            xidx, yidx, u_buf, ubg_buf, mb_buf, wv_buf, fcb_buf, out_buf,
            sem):
    c = lax.axis_index("c")
    s = lax.axis_index("s")
    wid = s * NC + c
    ibase = wid * I_PER

    copies = []
    for i in range(I_PER // CH):
        pltpu.sync_copy(xv.at[pl.ds(ibase + i * CH, CH)], xidx.at[i])
        copies.append(pltpu.async_copy(
            u_hbm.at[xidx.at[i]], u_buf.at[pl.ds(i * CH, CH)], sem))
        copies.append(pltpu.async_copy(
            ub_hbm.at[xidx.at[i]], ubg_buf.at[pl.ds(i * CH, CH)], sem))
        pltpu.sync_copy(yv.at[pl.ds(ibase + i * CH, CH)], yidx.at[i])
        copies.append(pltpu.async_copy(
            mb_hbm.at[yidx.at[i]], mb_buf.at[pl.ds(i * CH, CH)], sem))
    pltpu.sync_copy(vw_hbm.at[0], wv_buf)
    pltpu.sync_copy(fcb, fcb_buf)
    for cp in copies:
        cp.wait()
    w_splat = wv_buf[pl.ds(EMB, L)]
    fcb_splat = fcb_buf[...]

    def body(b, carry):
        z = (u_buf[pl.ds(b * L, L)]
             + w_splat * (ubg_buf[pl.ds(b * L, L)] + mb_buf[pl.ds(b * L, L)])
             + fcb_splat)
        out_buf[pl.ds(b * L, L)] = 1.0 / (1.0 + jnp.exp(-z))
        return carry

    lax.fori_loop(0, I_PER // L, body, 0)
    pltpu.sync_copy(out_buf, out_hbm.at[pl.ds(ibase, I_PER)])


@jax.jit
def kernel(x, y, usr_embd, usr_bias, mov_embd, mov_bias, fc_W, fc_b):
    xi = x.astype(jnp.int32)
    yi = y.astype(jnp.int32)
    mesh = plsc.VectorSubcoreMesh(core_axis_name="c", subcore_axis_name="s")
    sc_params = pltpu.CompilerParams(
        needs_layout_passes=False, use_tc_tiling_on_sc=False)

    scatter = functools.partial(
        pl.kernel,
        out_type=jax.ShapeDtypeStruct((NC * N_MOV_PAD,), jnp.float32),
        mesh=mesh,
        compiler_params=sc_params,
        scratch_types=[
            pltpu.VMEM((I_PER // CH, CH), jnp.int32),
            pltpu.VMEM((I_PER // CH, CH), jnp.float32),
            pltpu.VMEM((NSEG,), jnp.float32),
            pltpu.VMEM_SHARED((N_MOV_PAD,), jnp.float32),
        ],
    )(_sc_scatter)
    s_flat = scatter(yi, fc_W.reshape(-1))

    u_all, vw = pl.pallas_call(
        _tc_user,
        grid=(GRID,),
        in_specs=[
            pl.BlockSpec((EMB, BLK), lambda i: (0, i)),
            pl.BlockSpec((NC * N_MOV_PAD,), lambda i: (0,)),
            pl.BlockSpec((EMB, N_MOV), lambda i: (0, 0)),
        ],
        out_specs=[
            pl.BlockSpec((BLK,), lambda i: (i,)),
            pl.BlockSpec((1, 128), lambda i: (0, 0)),
        ],
        out_shape=[
            jax.ShapeDtypeStruct((N_PAD,), jnp.float32),
            jax.ShapeDtypeStruct((1, 128), jnp.float32),
        ],
        scratch_shapes=[pltpu.VMEM((1, 128), jnp.float32)],
    )(usr_embd.T, s_flat, mov_embd.T)

    final = functools.partial(
        pl.kernel,
        out_type=jax.ShapeDtypeStruct((B,), jnp.float32),
        mesh=mesh,
        compiler_params=sc_params,
        scratch_types=[
            pltpu.VMEM((I_PER // CH, CH), jnp.int32),
            pltpu.VMEM((I_PER // CH, CH), jnp.int32),
            pltpu.VMEM((I_PER,), jnp.float32),
            pltpu.VMEM((I_PER,), jnp.float32),
            pltpu.VMEM((I_PER,), jnp.float32),
            pltpu.VMEM((128,), jnp.float32),
            pltpu.VMEM((L,), jnp.float32),
            pltpu.VMEM((I_PER,), jnp.float32),
            pltpu.SemaphoreType.DMA,
        ],
    )(_sc_out)
    out = final(u_all, usr_bias.reshape(-1), mov_bias.reshape(-1), vw, xi, yi,
                jnp.broadcast_to(fc_b, (L,)))
    return out.reshape(B, 1)


# speedup vs baseline: 1.5682x; 1.5682x over previous
import functools

import jax
import jax.numpy as jnp
from jax import lax
from jax.experimental import pallas as pl
from jax.experimental.pallas import tpu as pltpu
from jax.experimental.pallas import tpu_sc as plsc

B = 16384
EMB = 16
L = 16
NC = 2
NS = 16
I_PER = B // (NC * NS)
CH = 128

N_USR = 1000001
N_MOV = 100001
NSEG = 6256
N_MOV_PAD = NS * NSEG
BLK = 131072
GRID = -(-N_USR // BLK)
N_PAD = GRID * BLK


def _sc_scatter(yv, fcw, s_out,
                yidx, w_buf, zero_buf, acc):
    c = lax.axis_index("c")
    s = lax.axis_index("s")
    wid = s * NC + c
    jbase = wid * I_PER

    zero = jnp.zeros((L,), jnp.float32)

    def zbody(b, carry):
        zero_buf[pl.ds(b * L, L)] = zero
        return carry

    lax.fori_loop(0, NSEG // L, zbody, 0)
    pltpu.sync_copy(zero_buf, acc.at[pl.ds(s * NSEG, NSEG)])
    plsc.subcore_barrier()

    for i in range(I_PER // CH):
        pltpu.sync_copy(yv.at[pl.ds(jbase + i * CH, CH)], yidx.at[i])
        pltpu.sync_copy(fcw.at[pl.ds(jbase + i * CH, CH)], w_buf.at[i])
        pltpu.sync_copy(w_buf.at[i], acc.at[yidx.at[i]], add=True)
    plsc.subcore_barrier()
    pltpu.sync_copy(acc.at[pl.ds(s * NSEG, NSEG)],
                    s_out.at[pl.ds(c * N_MOV_PAD + s * NSEG, NSEG)])


def _tc_user(ue_ref, ub_ref, s_ref, me_ref, out_ref, vw_out, vw_scr):
    @pl.when(pl.program_id(0) == 0)
    def _():
        s_sum = s_ref[0:N_MOV_PAD] + s_ref[N_MOV_PAD:2 * N_MOV_PAD]
        w_tot = jnp.sum(s_sum)
        v = jnp.sum(me_ref[...] * s_sum[0:N_MOV][None, :], axis=1)
        vw_scr[0, 0:EMB] = v
        vw_scr[0, EMB:2 * EMB] = jnp.zeros((EMB,), jnp.float32) + w_tot
        vw_scr[0, 2 * EMB:128] = jnp.zeros((128 - 2 * EMB,), jnp.float32)

    u = jax.lax.dot_general(vw_scr[0:1, 0:EMB], ue_ref[...],
                            (((1,), (0,)), ((), ())))
    out_ref[...] = u[0] + vw_scr[0, EMB] * ub_ref[0, :]
    vw_out[...] = vw_scr[...]


def _sc_out(u_hbm, mb_hbm, vw_hbm, xv, yv, fcb,
            out_hbm,
            xidx, yidx, u_buf, mb_buf, wv_buf, fcb_buf, out_buf, sem):
    c = lax.axis_index("c")
    s = lax.axis_index("s")
    wid = s * NC + c
    ibase = wid * I_PER

    copies = []
    for i in range(I_PER // CH):
        pltpu.sync_copy(xv.at[pl.ds(ibase + i * CH, CH)], xidx.at[i])
        copies.append(pltpu.async_copy(
            u_hbm.at[xidx.at[i]], u_buf.at[pl.ds(i * CH, CH)], sem))
        pltpu.sync_copy(yv.at[pl.ds(ibase + i * CH, CH)], yidx.at[i])
        copies.append(pltpu.async_copy(
            mb_hbm.at[yidx.at[i]], mb_buf.at[pl.ds(i * CH, CH)], sem))
    pltpu.sync_copy(vw_hbm.at[0], wv_buf)
    pltpu.sync_copy(fcb, fcb_buf)
    for cp in copies:
        cp.wait()
    w_splat = wv_buf[pl.ds(EMB, L)]
    fcb_splat = fcb_buf[...]

    def body(b, carry):
        z = (u_buf[pl.ds(b * L, L)]
             + w_splat * mb_buf[pl.ds(b * L, L)] + fcb_splat)
        out_buf[pl.ds(b * L, L)] = 1.0 / (1.0 + jnp.exp(-z))
        return carry

    lax.fori_loop(0, I_PER // L, body, 0)
    pltpu.sync_copy(out_buf, out_hbm.at[pl.ds(ibase, I_PER)])


@jax.jit
def kernel(x, y, usr_embd, usr_bias, mov_embd, mov_bias, fc_W, fc_b):
    xi = x.astype(jnp.int32)
    yi = y.astype(jnp.int32)
    mesh = plsc.VectorSubcoreMesh(core_axis_name="c", subcore_axis_name="s")
    sc_params = pltpu.CompilerParams(
        needs_layout_passes=False, use_tc_tiling_on_sc=False)

    scatter = functools.partial(
        pl.kernel,
        out_type=jax.ShapeDtypeStruct((NC * N_MOV_PAD,), jnp.float32),
        mesh=mesh,
        compiler_params=sc_params,
        scratch_types=[
            pltpu.VMEM((I_PER // CH, CH), jnp.int32),
            pltpu.VMEM((I_PER // CH, CH), jnp.float32),
            pltpu.VMEM((NSEG,), jnp.float32),
            pltpu.VMEM_SHARED((N_MOV_PAD,), jnp.float32),
        ],
    )(_sc_scatter)
    s_flat = scatter(yi, fc_W.reshape(-1))

    u_all, vw = pl.pallas_call(
        _tc_user,
        grid=(GRID,),
        in_specs=[
            pl.BlockSpec((EMB, BLK), lambda i: (0, i)),
            pl.BlockSpec((1, BLK), lambda i: (0, i)),
            pl.BlockSpec((NC * N_MOV_PAD,), lambda i: (0,)),
            pl.BlockSpec((EMB, N_MOV), lambda i: (0, 0)),
        ],
        out_specs=[
            pl.BlockSpec((BLK,), lambda i: (i,)),
            pl.BlockSpec((1, 128), lambda i: (0, 0)),
        ],
        out_shape=[
            jax.ShapeDtypeStruct((N_PAD,), jnp.float32),
            jax.ShapeDtypeStruct((1, 128), jnp.float32),
        ],
        scratch_shapes=[pltpu.VMEM((1, 128), jnp.float32)],
    )(usr_embd.T, usr_bias.T, s_flat, mov_embd.T)

    final = functools.partial(
        pl.kernel,
        out_type=jax.ShapeDtypeStruct((B,), jnp.float32),
        mesh=mesh,
        compiler_params=sc_params,
        scratch_types=[
            pltpu.VMEM((I_PER // CH, CH), jnp.int32),
            pltpu.VMEM((I_PER // CH, CH), jnp.int32),
            pltpu.VMEM((I_PER,), jnp.float32),
            pltpu.VMEM((I_PER,), jnp.float32),
            pltpu.VMEM((128,), jnp.float32),
            pltpu.VMEM((L,), jnp.float32),
            pltpu.VMEM((I_PER,), jnp.float32),
            pltpu.SemaphoreType.DMA,
        ],
    )(_sc_out)
    out = final(u_all, mov_bias.reshape(-1), vw, xi, yi,
                jnp.broadcast_to(fc_b, (L,)))
    return out.reshape(B, 1)


# speedup vs baseline: 1.6106x; 1.0270x over previous
import functools

import jax
import jax.numpy as jnp
from jax import lax
from jax.experimental import pallas as pl
from jax.experimental.pallas import tpu as pltpu
from jax.experimental.pallas import tpu_sc as plsc

B = 16384
EMB = 16
L = 16
NC = 2
NS = 16
I_PER = B // (NC * NS)
CH = 128

N_USR = 1000001
N_MOV = 100001
NSEG = 6272
N_MOV_PAD = NS * NSEG
BLK = 131072
GRID = -(-N_USR // BLK)
N_PAD = GRID * BLK


def _sc_scatter(yv, fcw, s_out,
                yidx, w_buf, zero_buf, acc):
    c = lax.axis_index("c")
    s = lax.axis_index("s")
    wid = s * NC + c
    jbase = wid * I_PER

    zero = jnp.zeros((L,), jnp.float32)

    def zbody(b, carry):
        for k in range(8):
            zero_buf[pl.ds((b * 8 + k) * L, L)] = zero
        return carry

    lax.fori_loop(0, NSEG // (8 * L), zbody, 0)
    pltpu.sync_copy(zero_buf, acc.at[pl.ds(s * NSEG, NSEG)])
    plsc.subcore_barrier()

    for i in range(I_PER // CH):
        pltpu.sync_copy(yv.at[pl.ds(jbase + i * CH, CH)], yidx.at[i])
        pltpu.sync_copy(fcw.at[pl.ds(jbase + i * CH, CH)], w_buf.at[i])
        pltpu.sync_copy(w_buf.at[i], acc.at[yidx.at[i]], add=True)
    plsc.subcore_barrier()
    pltpu.sync_copy(acc.at[pl.ds(s * NSEG, NSEG)],
                    s_out.at[pl.ds(c * N_MOV_PAD + s * NSEG, NSEG)])


def _tc_user(ue_ref, ub_ref, s_ref, me_ref, out_ref, vw_out, vw_scr):
    @pl.when(pl.program_id(0) == 0)
    def _():
        s_sum = s_ref[0:N_MOV_PAD] + s_ref[N_MOV_PAD:2 * N_MOV_PAD]
        w_tot = jnp.sum(s_sum)
        v = jnp.sum(me_ref[...] * s_sum[0:N_MOV][None, :], axis=1)
        vw_scr[0, 0:EMB] = v
        vw_scr[0, EMB:2 * EMB] = jnp.zeros((EMB,), jnp.float32) + w_tot
        vw_scr[0, 2 * EMB:128] = jnp.zeros((128 - 2 * EMB,), jnp.float32)

    u = jax.lax.dot_general(vw_scr[0:1, 0:EMB], ue_ref[...],
                            (((1,), (0,)), ((), ())))
    out_ref[...] = u[0] + vw_scr[0, EMB] * ub_ref[0, :]
    vw_out[...] = vw_scr[...]


def _sc_out(u_hbm, mb_hbm, vw_hbm, xv, yv, fcb,
            out_hbm,
            xidx, yidx, u_buf, mb_buf, wv_buf, fcb_buf, out_buf, sem):
    c = lax.axis_index("c")
    s = lax.axis_index("s")
    wid = s * NC + c
    ibase = wid * I_PER

    copies = []
    for i in range(I_PER // CH):
        pltpu.sync_copy(xv.at[pl.ds(ibase + i * CH, CH)], xidx.at[i])
        copies.append(pltpu.async_copy(
            u_hbm.at[xidx.at[i]], u_buf.at[pl.ds(i * CH, CH)], sem))
        pltpu.sync_copy(yv.at[pl.ds(ibase + i * CH, CH)], yidx.at[i])
        copies.append(pltpu.async_copy(
            mb_hbm.at[yidx.at[i]], mb_buf.at[pl.ds(i * CH, CH)], sem))
    pltpu.sync_copy(vw_hbm.at[0], wv_buf)
    pltpu.sync_copy(fcb, fcb_buf)
    for cp in copies:
        cp.wait()
    w_splat = wv_buf[pl.ds(EMB, L)]
    fcb_splat = fcb_buf[...]

    def body(b, carry):
        z = (u_buf[pl.ds(b * L, L)]
             + w_splat * mb_buf[pl.ds(b * L, L)] + fcb_splat)
        out_buf[pl.ds(b * L, L)] = 1.0 / (1.0 + jnp.exp(-z))
        return carry

    lax.fori_loop(0, I_PER // L, body, 0)
    pltpu.sync_copy(out_buf, out_hbm.at[pl.ds(ibase, I_PER)])


@jax.jit
def kernel(x, y, usr_embd, usr_bias, mov_embd, mov_bias, fc_W, fc_b):
    xi = x.astype(jnp.int32)
    yi = y.astype(jnp.int32)
    mesh = plsc.VectorSubcoreMesh(core_axis_name="c", subcore_axis_name="s")
    sc_params = pltpu.CompilerParams(
        needs_layout_passes=False, use_tc_tiling_on_sc=False)

    scatter = functools.partial(
        pl.kernel,
        out_type=jax.ShapeDtypeStruct((NC * N_MOV_PAD,), jnp.float32),
        mesh=mesh,
        compiler_params=sc_params,
        scratch_types=[
            pltpu.VMEM((I_PER // CH, CH), jnp.int32),
            pltpu.VMEM((I_PER // CH, CH), jnp.float32),
            pltpu.VMEM((NSEG,), jnp.float32),
            pltpu.VMEM_SHARED((N_MOV_PAD,), jnp.float32),
        ],
    )(_sc_scatter)
    s_flat = scatter(yi, fc_W.reshape(-1))

    u_all, vw = pl.pallas_call(
        _tc_user,
        grid=(GRID,),
        in_specs=[
            pl.BlockSpec((EMB, BLK), lambda i: (0, i)),
            pl.BlockSpec((1, BLK), lambda i: (0, i)),
            pl.BlockSpec((NC * N_MOV_PAD,), lambda i: (0,)),
            pl.BlockSpec((EMB, N_MOV), lambda i: (0, 0)),
        ],
        out_specs=[
            pl.BlockSpec((BLK,), lambda i: (i,)),
            pl.BlockSpec((1, 128), lambda i: (0, 0)),
        ],
        out_shape=[
            jax.ShapeDtypeStruct((N_PAD,), jnp.float32),
            jax.ShapeDtypeStruct((1, 128), jnp.float32),
        ],
        scratch_shapes=[pltpu.VMEM((1, 128), jnp.float32)],
    )(usr_embd.T, usr_bias.T, s_flat, mov_embd.T)

    final = functools.partial(
        pl.kernel,
        out_type=jax.ShapeDtypeStruct((B,), jnp.float32),
        mesh=mesh,
        compiler_params=sc_params,
        scratch_types=[
            pltpu.VMEM((I_PER // CH, CH), jnp.int32),
            pltpu.VMEM((I_PER // CH, CH), jnp.int32),
            pltpu.VMEM((I_PER,), jnp.float32),
            pltpu.VMEM((I_PER,), jnp.float32),
            pltpu.VMEM((128,), jnp.float32),
            pltpu.VMEM((L,), jnp.float32),
            pltpu.VMEM((I_PER,), jnp.float32),
            pltpu.SemaphoreType.DMA,
        ],
    )(_sc_out)
    out = final(u_all, mov_bias.reshape(-1), vw, xi, yi,
                jnp.broadcast_to(fc_b, (L,)))
    return out.reshape(B, 1)
